# baseline (speedup 1.0000x reference)
import functools

import jax
import jax.numpy as jnp
from jax import lax
from jax.experimental import pallas as pl
from jax.experimental.pallas import tpu as pltpu
from jax.experimental.pallas import tpu_sc as plsc

_VOCAB = 1025
_HID = 768
_NTOK = 16384

_K = 8192
_M = 512

_NC = 2
_NS = 16
_NW = _NC * _NS

_B_PER_W = (_NTOK - _K) // _NW
_CHUNK = 32
_DEPTH = 4
_N_CHUNKS = _B_PER_W // _CHUNK


@functools.cache
def _build_sc():
    mesh = plsc.VectorSubcoreMesh(core_axis_name="c", subcore_axis_name="s")

    @functools.partial(
        pl.kernel,
        mesh=mesh,
        out_type=jax.ShapeDtypeStruct((_NTOK, _HID), jnp.float32),
        scratch_types=[
            pltpu.VMEM((_B_PER_W,), jnp.int32),
            pltpu.VMEM((_DEPTH, _CHUNK, _HID), jnp.float32),
        ] + [pltpu.SemaphoreType.DMA] * (2 * _DEPTH),
    )
    def gather_kernel(table_hbm, idx_hbm, out_hbm, idx_v, rows_v, *sems):
        wid = lax.axis_index("s") * _NC + lax.axis_index("c")
        base = _K + wid * _B_PER_W
        pltpu.sync_copy(idx_hbm.at[pl.ds(base, _B_PER_W)], idx_v)

        gsems = sems[:_DEPTH]
        ssems = sems[_DEPTH:]
        gathers = [None] * _DEPTH
        scatters = [None] * _DEPTH

        for i in range(_DEPTH - 1):
            gathers[i] = pltpu.async_copy(
                table_hbm.at[idx_v.at[pl.ds(i * _CHUNK, _CHUNK)]],
                rows_v.at[i], gsems[i])
        for i in range(_N_CHUNKS):
            buf = i % _DEPTH
            nxt = i + _DEPTH - 1
            if nxt < _N_CHUNKS:
                nbuf = nxt % _DEPTH
                if scatters[nbuf] is not None:
                    scatters[nbuf].wait()
                    scatters[nbuf] = None
                gathers[nbuf] = pltpu.async_copy(
                    table_hbm.at[idx_v.at[pl.ds(nxt * _CHUNK, _CHUNK)]],
                    rows_v.at[nbuf], gsems[nbuf])
            gathers[buf].wait()
            scatters[buf] = pltpu.async_copy(
                rows_v.at[buf],
                out_hbm.at[pl.ds(base + i * _CHUNK, _CHUNK)],
                ssems[buf])
        for s in scatters:
            if s is not None:
                s.wait()

    return gather_kernel


def _tc_body(ids_ref, table_ref, out_ref, hi_ref, lo_ref):
    i = pl.program_id(0)

    @pl.when(i == 0)
    def _():
        tab = table_ref[...]
        hi = tab.astype(jnp.bfloat16)
        hi_ref[...] = hi
        lo_ref[...] = (tab - hi.astype(jnp.float32)).astype(jnp.bfloat16)

    ids = ids_ref[...]
    vocab_iota = lax.broadcasted_iota(jnp.int32, (_M, _VOCAB), 1)
    onehot = (vocab_iota == ids).astype(jnp.bfloat16)
    out_ref[...] = (
        jnp.dot(onehot, hi_ref[...], preferred_element_type=jnp.float32)
        + jnp.dot(onehot, lo_ref[...], preferred_element_type=jnp.float32))


@functools.cache
def _build_tc():
    return pl.pallas_call(
        _tc_body,
        grid=(_K // _M,),
        in_specs=[
            pl.BlockSpec((_M, 1), lambda i: (i, 0)),
            pl.BlockSpec((_VOCAB, _HID), lambda i: (0, 0)),
        ],
        out_specs=pl.BlockSpec((_M, _HID), lambda i: (i, 0)),
        out_shape=jax.ShapeDtypeStruct((_K, _HID), jnp.float32),
        scratch_shapes=[
            pltpu.VMEM((_VOCAB, _HID), jnp.bfloat16),
            pltpu.VMEM((_VOCAB, _HID), jnp.bfloat16),
        ],
    )


def kernel(input_ids, positions, embed_tokens):
    del positions
    ids = input_ids.astype(jnp.int32)
    sc_out = _build_sc()(embed_tokens, ids)
    tc_out = _build_tc()(ids[:_K].reshape(_K, 1), embed_tokens)
    return lax.dynamic_update_slice(sc_out, tc_out, (0, 0))


# speedup vs baseline: 1.3200x; 1.3200x over previous
import functools

import jax
import jax.numpy as jnp
from jax import lax
from jax.experimental import pallas as pl
from jax.experimental.pallas import tpu as pltpu
from jax.experimental.pallas import tpu_sc as plsc

_VOCAB = 1025
_HID = 768
_NTOK = 16384

_NC = 2
_NS = 16
_NW = _NC * _NS

_B_PER_W = _NTOK // _NW
_CHUNK = 32
_DEPTH = 4
_N_CHUNKS = _B_PER_W // _CHUNK


@functools.cache
def _build():
    mesh = plsc.VectorSubcoreMesh(core_axis_name="c", subcore_axis_name="s")

    @functools.partial(
        pl.kernel,
        mesh=mesh,
        out_type=jax.ShapeDtypeStruct((_NTOK, _HID), jnp.float32),
        scratch_types=[
            pltpu.VMEM((_B_PER_W,), jnp.int32),
            pltpu.VMEM((_DEPTH, _CHUNK, _HID), jnp.float32),
        ] + [pltpu.SemaphoreType.DMA] * (2 * _DEPTH),
    )
    def gather_kernel(table_hbm, idx_hbm, out_hbm, idx_v, rows_v, *sems):
        wid = lax.axis_index("s") * _NC + lax.axis_index("c")
        base = wid * _B_PER_W
        pltpu.sync_copy(idx_hbm.at[pl.ds(base, _B_PER_W)], idx_v)

        gsems = sems[:_DEPTH]
        ssems = sems[_DEPTH:]
        gathers = [None] * _DEPTH
        scatters = [None] * _DEPTH

        for i in range(_DEPTH - 1):
            gathers[i] = pltpu.async_copy(
                table_hbm.at[idx_v.at[pl.ds(i * _CHUNK, _CHUNK)]],
                rows_v.at[i], gsems[i])
        for i in range(_N_CHUNKS):
            buf = i % _DEPTH
            nxt = i + _DEPTH - 1
            if nxt < _N_CHUNKS:
                nbuf = nxt % _DEPTH
                if scatters[nbuf] is not None:
                    scatters[nbuf].wait()
                    scatters[nbuf] = None
                gathers[nbuf] = pltpu.async_copy(
                    table_hbm.at[idx_v.at[pl.ds(nxt * _CHUNK, _CHUNK)]],
                    rows_v.at[nbuf], gsems[nbuf])
            gathers[buf].wait()
            scatters[buf] = pltpu.async_copy(
                rows_v.at[buf], out_hbm.at[pl.ds(base + i * _CHUNK, _CHUNK)],
                ssems[buf])
        for s in scatters:
            if s is not None:
                s.wait()

    return gather_kernel


def kernel(input_ids, positions, embed_tokens):
    del positions
    return _build()(embed_tokens, input_ids.astype(jnp.int32))
